# Initial kernel scaffold; baseline (speedup 1.0000x reference)
import jax
import jax.numpy as jnp
from jax.experimental import pallas as pl


def kernel(signatures, selector, emb_app_c, emb_tf_c, emb_app_l, emb_tf_l, W1c, b1c, W2c, b2c, W1l, b1l, W2l, b2l):
    raise NotImplementedError("write your pallas kernel here")


# speedup vs baseline: 8.5125x; 8.5125x over previous
import functools

import jax
import jax.numpy as jnp
from jax.experimental import pallas as pl
from jax.experimental.pallas import tpu as pltpu

N = 8192
L = 64
H = 2048
B = 256
HB = 512
NB = N // B
NH = H // HB
NQ = 12


def _quantize(s):
    return jnp.where(s < 1e-8,
                     jnp.zeros(s.shape, jnp.int32),
                     jnp.floor(s * 10.0).astype(jnp.int32) + 1)


def _embed_dim(q, table, dim):
    acc = jnp.zeros(q.shape, jnp.float32)
    for k in range(NQ):
        acc = acc + jnp.where(q == k, table[k, dim], 0.0)
    return acc


def _mlp_block(sa, st, ea, et, W1r, b1, W2blk, b2blk):
    qa = _quantize(sa)
    qt = _quantize(st)
    e0 = _embed_dim(qa, ea, 0)
    e1 = _embed_dim(qa, ea, 1)
    t0 = _embed_dim(qt, et, 0)
    t1 = _embed_dim(qt, et, 1)
    dot = functools.partial(jnp.dot, preferred_element_type=jnp.float32)
    h = dot(e0, W1r[0]) + dot(e1, W1r[1]) + dot(t0, W1r[2]) + dot(t1, W1r[3])
    h = jnp.maximum(h + b1, 0.0)
    return dot(h, W2blk) + b2blk


def _body(sig_ref, sel_ref, eac, etc, eal, etl,
          W1c_ref, b1c_ref, W2c_ref, b2c_ref,
          W1l_ref, b1l_ref, W2l_ref, b2l_ref, out_ref):
    sa = sig_ref[:, :L]
    st = sig_ref[:, L:]
    yc = _mlp_block(sa, st, eac[...], etc[...], W1c_ref[...], b1c_ref[...],
                    W2c_ref[...], b2c_ref[...])
    yl = _mlp_block(sa, st, eal[...], etl[...], W1l_ref[...], b1l_ref[...],
                    W2l_ref[...], b2l_ref[...])
    sel = sel_ref[...]
    out_ref[...] = jnp.where(sel > 0, yl, yc)


def kernel(signatures, selector, emb_app_c, emb_tf_c, emb_app_l, emb_tf_l,
           W1c, b1c, W2c, b2c, W1l, b1l, W2l, b2l):
    sig2 = jnp.concatenate([signatures[:, :, 0], signatures[:, :, 1]], axis=1)
    sel2 = selector.reshape(N, 1)
    W1cr = W1c.reshape(L, 4, 2 * H).transpose(1, 0, 2)
    W1lr = W1l.reshape(L, 4, 2 * H).transpose(1, 0, 2)
    b1c2 = b1c.reshape(1, 2 * H)
    b1l2 = b1l.reshape(1, 2 * H)
    b2c2 = b2c.reshape(1, H)
    b2l2 = b2l.reshape(1, H)

    full = lambda shape: pl.BlockSpec(shape, lambda nh, it: (0,) * len(shape))
    grid_spec = pltpu.PrefetchScalarGridSpec(
        num_scalar_prefetch=0,
        grid=(NH, NB),
        in_specs=[
            pl.BlockSpec((B, 2 * L), lambda nh, it: (it, 0)),
            pl.BlockSpec((B, 1), lambda nh, it: (it, 0)),
            full((NQ, 2)), full((NQ, 2)), full((NQ, 2)), full((NQ, 2)),
            full((4, L, 2 * H)),
            full((1, 2 * H)),
            pl.BlockSpec((2 * H, HB), lambda nh, it: (0, nh)),
            pl.BlockSpec((1, HB), lambda nh, it: (0, nh)),
            full((4, L, 2 * H)),
            full((1, 2 * H)),
            pl.BlockSpec((2 * H, HB), lambda nh, it: (0, nh)),
            pl.BlockSpec((1, HB), lambda nh, it: (0, nh)),
        ],
        out_specs=pl.BlockSpec((B, HB), lambda nh, it: (it, nh)),
    )
    return pl.pallas_call(
        _body,
        grid_spec=grid_spec,
        out_shape=jax.ShapeDtypeStruct((N, H), jnp.float32),
    )(sig2, sel2, emb_app_c, emb_tf_c, emb_app_l, emb_tf_l,
      W1cr, b1c2, W2c, b2c2, W1lr, b1l2, W2l, b2l2)


# speedup vs baseline: 15.8503x; 1.8620x over previous
import functools

import jax
import jax.numpy as jnp
from jax import lax
from jax.experimental import pallas as pl
from jax.experimental.pallas import tpu as pltpu
from jax.experimental.pallas import tpu_sc as plsc

N = 8192
L = 64
H = 2048
B = 256
NPAD = N + B
NBP = NPAD // B
NH = 2
HB = H // NH
NQ = 12

NC, NS = 2, 16
NW = NC * NS


def _make_row_gather(n_rows, n_cols, dtype, chunk):
    per_w = n_rows // NW
    assert per_w % chunk == 0 and chunk % 8 == 0 and chunk <= 128
    n_chunks = per_w // chunk
    mesh = plsc.VectorSubcoreMesh(core_axis_name="c", subcore_axis_name="s")

    def body(table_hbm, idx_hbm, out_hbm, idx_v, rows_v, sem):
        wid = lax.axis_index("s") * NC + lax.axis_index("c")
        base = wid * per_w
        for j in range(n_chunks):
            off = base + j * chunk
            pltpu.sync_copy(idx_hbm.at[pl.ds(off, chunk)], idx_v)
            pltpu.async_copy(table_hbm.at[idx_v], rows_v, sem).wait()
            pltpu.sync_copy(rows_v, out_hbm.at[pl.ds(off, chunk)])

    return pl.kernel(
        body,
        mesh=mesh,
        out_type=jax.ShapeDtypeStruct((n_rows, n_cols), dtype),
        scratch_types=[
            pltpu.VMEM((chunk,), jnp.int32),
            pltpu.VMEM((chunk, n_cols), dtype),
            pltpu.SemaphoreType.DMA,
        ],
    )


def _quantize(s):
    return jnp.where(s < 1e-8,
                     jnp.zeros(s.shape, jnp.int32),
                     jnp.floor(s * 10.0).astype(jnp.int32) + 1)


def _embed_dim(q, table, dim):
    acc = jnp.zeros(q.shape, jnp.float32)
    for k in range(NQ):
        acc = acc + jnp.where(q == k, table[0, k, dim], 0.0)
    return acc


def _mlp_body(eid_ref, sig_ref, ea_ref, et_ref, W1_ref, b1_ref, W2_ref, b2_ref,
              out_ref):
    sa = sig_ref[:, :L]
    st = sig_ref[:, L:]
    qa = _quantize(sa)
    qt = _quantize(st)
    ea = ea_ref[...]
    et = et_ref[...]
    e0 = _embed_dim(qa, ea, 0)
    e1 = _embed_dim(qa, ea, 1)
    t0 = _embed_dim(qt, et, 0)
    t1 = _embed_dim(qt, et, 1)
    dot = functools.partial(jnp.dot, preferred_element_type=jnp.float32)
    W1 = W1_ref[0]
    h = dot(e0, W1[0]) + dot(e1, W1[1]) + dot(t0, W1[2]) + dot(t1, W1[3])
    h = jnp.maximum(h + b1_ref[0], 0.0)
    out_ref[...] = dot(h, W2_ref[0]) + b2_ref[0]


def kernel(signatures, selector, emb_app_c, emb_tf_c, emb_app_l, emb_tf_l,
           W1c, b1c, W2c, b2c, W1l, b1l, W2l, b2l):
    is_c = (selector == 0).astype(jnp.int32)
    r0 = jnp.cumsum(is_c) - is_c
    r1 = jnp.cumsum(1 - is_c) - (1 - is_c)
    n0 = jnp.sum(is_c)
    n0p = ((n0 + B - 1) // B) * B
    row = jnp.where(is_c > 0, r0, n0p + r1)
    src = jnp.zeros((NPAD,), jnp.int32).at[row].set(
        jnp.arange(N, dtype=jnp.int32))
    eid = (jnp.arange(NBP, dtype=jnp.int32) * B >= n0p).astype(jnp.int32)

    sig2 = jnp.concatenate([signatures[:, :, 0], signatures[:, :, 1]], axis=1)
    sig_sorted = _make_row_gather(NPAD, 2 * L, jnp.float32, chunk=88)(sig2, src)

    W1s = jnp.stack([W1c, W1l]).reshape(2, L, 4, 2 * H).transpose(0, 2, 1, 3)
    W2s = jnp.stack([W2c, W2l])
    b1s = jnp.stack([b1c, b1l]).reshape(2, 1, 2 * H)
    b2s = jnp.stack([b2c, b2l]).reshape(2, 1, H)
    eas = jnp.stack([emb_app_c, emb_app_l])
    ets = jnp.stack([emb_tf_c, emb_tf_l])

    grid_spec = pltpu.PrefetchScalarGridSpec(
        num_scalar_prefetch=1,
        grid=(NH, NBP),
        in_specs=[
            pl.BlockSpec((B, 2 * L), lambda nh, it, eid: (it, 0)),
            pl.BlockSpec((1, NQ, 2), lambda nh, it, eid: (eid[it], 0, 0)),
            pl.BlockSpec((1, NQ, 2), lambda nh, it, eid: (eid[it], 0, 0)),
            pl.BlockSpec((1, 4, L, 2 * H), lambda nh, it, eid: (eid[it], 0, 0, 0)),
            pl.BlockSpec((1, 1, 2 * H), lambda nh, it, eid: (eid[it], 0, 0)),
            pl.BlockSpec((1, 2 * H, HB), lambda nh, it, eid: (eid[it], 0, nh)),
            pl.BlockSpec((1, 1, HB), lambda nh, it, eid: (eid[it], 0, nh)),
        ],
        out_specs=pl.BlockSpec((B, HB), lambda nh, it, eid: (it, nh)),
    )
    y_sorted = pl.pallas_call(
        _mlp_body,
        grid_spec=grid_spec,
        out_shape=jax.ShapeDtypeStruct((NPAD, H), jnp.float32),
    )(eid, sig_sorted, eas, ets, W1s, b1s, W2s, b2s)

    return _make_row_gather(N, H, jnp.float32, chunk=32)(y_sorted, row)


# speedup vs baseline: 19.2860x; 1.2168x over previous
import functools

import jax
import jax.numpy as jnp
from jax import lax
from jax.experimental import pallas as pl
from jax.experimental.pallas import tpu as pltpu
from jax.experimental.pallas import tpu_sc as plsc

N = 8192
L = 64
H = 2048
B = 256
NPAD = N + B
NBP = NPAD // B
NH = 1
HB = H // NH
NQ = 12

NC, NS = 2, 16
NW = NC * NS


def _make_row_gather(n_rows, n_cols, dtype, chunk):
    per_w = n_rows // NW
    assert per_w % chunk == 0 and chunk % 8 == 0 and chunk <= 128
    n_chunks = per_w // chunk
    mesh = plsc.VectorSubcoreMesh(core_axis_name="c", subcore_axis_name="s")

    def body(table_hbm, idx_hbm, out_hbm, idx_v, rows_v, sem):
        wid = lax.axis_index("s") * NC + lax.axis_index("c")
        base = wid * per_w
        for j in range(n_chunks):
            off = base + j * chunk
            pltpu.sync_copy(idx_hbm.at[pl.ds(off, chunk)], idx_v)
            pltpu.async_copy(table_hbm.at[idx_v], rows_v, sem).wait()
            pltpu.sync_copy(rows_v, out_hbm.at[pl.ds(off, chunk)])

    return pl.kernel(
        body,
        mesh=mesh,
        out_type=jax.ShapeDtypeStruct((n_rows, n_cols), dtype),
        scratch_types=[
            pltpu.VMEM((chunk,), jnp.int32),
            pltpu.VMEM((chunk, n_cols), dtype),
            pltpu.SemaphoreType.DMA,
        ],
    )


def _quantize(s):
    return jnp.where(s < 1e-8,
                     jnp.zeros(s.shape, jnp.int32),
                     jnp.floor(s * 10.0).astype(jnp.int32) + 1)


def _embed_dim(q, table, dim):
    acc = jnp.zeros(q.shape, jnp.float32)
    for k in range(NQ):
        acc = acc + jnp.where(q == k, table[0, k, dim], 0.0)
    return acc


def _mlp_body(eid_ref, sig_ref, ea_ref, et_ref, W1_ref, b1_ref, W2_ref, b2_ref,
              out_ref):
    sa = sig_ref[:, :L]
    st = sig_ref[:, L:]
    qa = _quantize(sa)
    qt = _quantize(st)
    ea = ea_ref[...]
    et = et_ref[...]
    bf = jnp.bfloat16
    e0 = _embed_dim(qa, ea, 0).astype(bf)
    e1 = _embed_dim(qa, ea, 1).astype(bf)
    t0 = _embed_dim(qt, et, 0).astype(bf)
    t1 = _embed_dim(qt, et, 1).astype(bf)
    dot = functools.partial(jnp.dot, preferred_element_type=jnp.float32)
    W1 = W1_ref[0]
    h = dot(e0, W1[0]) + dot(e1, W1[1]) + dot(t0, W1[2]) + dot(t1, W1[3])
    h = jnp.maximum(h + b1_ref[0], 0.0).astype(bf)
    out_ref[...] = dot(h, W2_ref[0]) + b2_ref[0]


def kernel(signatures, selector, emb_app_c, emb_tf_c, emb_app_l, emb_tf_l,
           W1c, b1c, W2c, b2c, W1l, b1l, W2l, b2l):
    is_c = (selector == 0).astype(jnp.int32)
    r0 = jnp.cumsum(is_c) - is_c
    r1 = jnp.cumsum(1 - is_c) - (1 - is_c)
    n0 = jnp.sum(is_c)
    n0p = ((n0 + B - 1) // B) * B
    row = jnp.where(is_c > 0, r0, n0p + r1)
    src = jnp.zeros((NPAD,), jnp.int32).at[row].set(
        jnp.arange(N, dtype=jnp.int32))
    eid = (jnp.arange(NBP, dtype=jnp.int32) * B >= n0p).astype(jnp.int32)

    sig2 = jnp.concatenate([signatures[:, :, 0], signatures[:, :, 1]], axis=1)
    sig_sorted = _make_row_gather(NPAD, 2 * L, jnp.float32, chunk=88)(sig2, src)

    W1s = (jnp.stack([W1c, W1l]).reshape(2, L, 4, 2 * H)
           .transpose(0, 2, 1, 3).astype(jnp.bfloat16))
    W2s = jnp.stack([W2c, W2l]).astype(jnp.bfloat16)
    b1s = jnp.stack([b1c, b1l]).reshape(2, 1, 2 * H)
    b2s = jnp.stack([b2c, b2l]).reshape(2, 1, H)
    eas = jnp.stack([emb_app_c, emb_app_l])
    ets = jnp.stack([emb_tf_c, emb_tf_l])

    grid_spec = pltpu.PrefetchScalarGridSpec(
        num_scalar_prefetch=1,
        grid=(NH, NBP),
        in_specs=[
            pl.BlockSpec((B, 2 * L), lambda nh, it, eid: (it, 0)),
            pl.BlockSpec((1, NQ, 2), lambda nh, it, eid: (eid[it], 0, 0)),
            pl.BlockSpec((1, NQ, 2), lambda nh, it, eid: (eid[it], 0, 0)),
            pl.BlockSpec((1, 4, L, 2 * H), lambda nh, it, eid: (eid[it], 0, 0, 0)),
            pl.BlockSpec((1, 1, 2 * H), lambda nh, it, eid: (eid[it], 0, 0)),
            pl.BlockSpec((1, 2 * H, HB), lambda nh, it, eid: (eid[it], 0, nh)),
            pl.BlockSpec((1, 1, HB), lambda nh, it, eid: (eid[it], 0, nh)),
        ],
        out_specs=pl.BlockSpec((B, HB), lambda nh, it, eid: (it, nh)),
    )
    y_sorted = pl.pallas_call(
        _mlp_body,
        grid_spec=grid_spec,
        out_shape=jax.ShapeDtypeStruct((NPAD, H), jnp.float32),
    )(eid, sig_sorted, eas, ets, W1s, b1s, W2s, b2s)

    return _make_row_gather(N, H, jnp.float32, chunk=32)(y_sorted, row)


# speedup vs baseline: 21.1473x; 1.0965x over previous
import functools

import jax
import jax.numpy as jnp
from jax import lax
from jax.experimental import pallas as pl
from jax.experimental.pallas import tpu as pltpu
from jax.experimental.pallas import tpu_sc as plsc

N = 8192
L = 64
H = 2048
IN = 2 * L
B = 512
NPAD = N + B
NBP = NPAD // B
NQ = 12

NC, NS = 2, 16
NW = NC * NS


def _make_row_gather(n_rows, n_cols, dtype, chunk):
    per_w = n_rows // NW
    assert per_w % chunk == 0 and chunk % 8 == 0 and chunk <= 128
    n_chunks = per_w // chunk
    mesh = plsc.VectorSubcoreMesh(core_axis_name="c", subcore_axis_name="s")

    def body(table_hbm, idx_hbm, out_hbm, idx_v, rows_v, sem):
        wid = lax.axis_index("s") * NC + lax.axis_index("c")
        base = wid * per_w
        for j in range(n_chunks):
            off = base + j * chunk
            pltpu.sync_copy(idx_hbm.at[pl.ds(off, chunk)], idx_v)
            pltpu.async_copy(table_hbm.at[idx_v], rows_v, sem).wait()
            pltpu.sync_copy(rows_v, out_hbm.at[pl.ds(off, chunk)])

    return pl.kernel(
        body,
        mesh=mesh,
        out_type=jax.ShapeDtypeStruct((n_rows, n_cols), dtype),
        scratch_types=[
            pltpu.VMEM((chunk,), jnp.int32),
            pltpu.VMEM((chunk, n_cols), dtype),
            pltpu.SemaphoreType.DMA,
        ],
    )


def _embed(q, parity, ea, et, dim):
    acc = jnp.zeros(q.shape, jnp.float32)
    for k in range(NQ):
        val = jnp.where(parity > 0, et[k, dim], ea[k, dim])
        acc = acc + jnp.where(q == k, val, 0.0)
    return acc.astype(jnp.bfloat16)


def _expert_mlp(x0, x1, W1_ref, b1_ref, W2_ref, b2_ref):
    dot = functools.partial(jnp.dot, preferred_element_type=jnp.float32)
    h = dot(x0, W1_ref[0]) + dot(x1, W1_ref[1]) + b1_ref[...]
    h = jnp.maximum(h, 0.0).astype(jnp.bfloat16)
    return dot(h, W2_ref[...]) + b2_ref[...]


def _mlp_body(eid_ref, sig_ref, eac, etc, eal, etl,
              W1c_ref, b1c_ref, W2c_ref, b2c_ref,
              W1l_ref, b1l_ref, W2l_ref, b2l_ref, out_ref):
    s = sig_ref[...]
    q = jnp.where(s < 1e-8,
                  jnp.zeros(s.shape, jnp.int32),
                  jnp.floor(s * 10.0).astype(jnp.int32) + 1)
    parity = lax.broadcasted_iota(jnp.int32, (1, IN), 1) % 2
    e = eid_ref[pl.program_id(0)]

    ea = jnp.where(e > 0, eal[...], eac[...])
    et = jnp.where(e > 0, etl[...], etc[...])
    x0 = _embed(q, parity, ea, et, 0)
    x1 = _embed(q, parity, ea, et, 1)

    @pl.when(e == 0)
    def _():
        out_ref[...] = _expert_mlp(x0, x1, W1c_ref, b1c_ref, W2c_ref, b2c_ref)

    @pl.when(e != 0)
    def _():
        out_ref[...] = _expert_mlp(x0, x1, W1l_ref, b1l_ref, W2l_ref, b2l_ref)


def _w1_split(W1):
    W = W1.reshape(L, 2, 2, 2 * H)
    WA = W[:, :, 0, :].reshape(IN, 2 * H)
    WB = W[:, :, 1, :].reshape(IN, 2 * H)
    return jnp.stack([WA, WB]).astype(jnp.bfloat16)


def kernel(signatures, selector, emb_app_c, emb_tf_c, emb_app_l, emb_tf_l,
           W1c, b1c, W2c, b2c, W1l, b1l, W2l, b2l):
    is_c = (selector == 0).astype(jnp.int32)
    r0 = jnp.cumsum(is_c) - is_c
    r1 = jnp.cumsum(1 - is_c) - (1 - is_c)
    n0 = jnp.sum(is_c)
    n0p = ((n0 + B - 1) // B) * B
    row = jnp.where(is_c > 0, r0, n0p + r1)
    src = jnp.zeros((NPAD,), jnp.int32).at[row].set(
        jnp.arange(N, dtype=jnp.int32))
    eid = (jnp.arange(NBP, dtype=jnp.int32) * B >= n0p).astype(jnp.int32)

    sig2 = signatures.reshape(N, IN)
    sig_sorted = _make_row_gather(NPAD, IN, jnp.float32, chunk=16)(sig2, src)

    W1cs, W1ls = _w1_split(W1c), _w1_split(W1l)
    W2cb = W2c.astype(jnp.bfloat16)
    W2lb = W2l.astype(jnp.bfloat16)
    b1c2 = b1c.reshape(1, 2 * H)
    b1l2 = b1l.reshape(1, 2 * H)
    b2c2 = b2c.reshape(1, H)
    b2l2 = b2l.reshape(1, H)

    full = lambda shape: pl.BlockSpec(shape, lambda it, eid: (0,) * len(shape))
    grid_spec = pltpu.PrefetchScalarGridSpec(
        num_scalar_prefetch=1,
        grid=(NBP,),
        in_specs=[
            pl.BlockSpec((B, IN), lambda it, eid: (it, 0)),
            full((NQ, 2)), full((NQ, 2)), full((NQ, 2)), full((NQ, 2)),
            full((2, IN, 2 * H)), full((1, 2 * H)),
            full((2 * H, H)), full((1, H)),
            full((2, IN, 2 * H)), full((1, 2 * H)),
            full((2 * H, H)), full((1, H)),
        ],
        out_specs=pl.BlockSpec((B, H), lambda it, eid: (it, 0)),
    )
    y_sorted = pl.pallas_call(
        _mlp_body,
        grid_spec=grid_spec,
        out_shape=jax.ShapeDtypeStruct((NPAD, H), jnp.float32),
    )(eid, sig_sorted, emb_app_c, emb_tf_c, emb_app_l, emb_tf_l,
      W1cs, b1c2, W2cb, b2c2, W1ls, b1l2, W2lb, b2l2)

    return _make_row_gather(N, H, jnp.float32, chunk=32)(y_sorted, row)
